# baseline (device time: 13412 ns/iter reference)
import functools

import jax
import jax.numpy as jnp
from jax import lax
from jax.experimental import pallas as pl
from jax.experimental.pallas import tpu as pltpu

N_CHUNKS = 8


def kernel(partial, gamma):
    m_half = partial.shape[1] // 2
    quarter = m_half // 2
    d = partial.shape[2]
    ch = quarter // N_CHUNKS

    def body(
        p_ref, g_ref, out_ref,
        sstage, lbuf, obuf, ysend, yrecv, xrecv,
        xbar_sem, local_sems, ysend_sems, yrecv_sems, xsend_sems,
        xrecv_sems, out_sems,
    ):
        my_x = lax.axis_index("x")
        my_y = lax.axis_index("y")
        my_z = lax.axis_index("z")
        y_nbr = (my_x, 1 - my_y, my_z)
        x_nbr = (1 - my_x, my_y, my_z)

        barrier_sem = pltpu.get_barrier_semaphore()
        pl.semaphore_signal(
            barrier_sem, inc=1, device_id=y_nbr,
            device_id_type=pl.DeviceIdType.MESH,
        )
        pl.semaphore_signal(
            xbar_sem, inc=1, device_id=x_nbr,
            device_id_type=pl.DeviceIdType.MESH,
        )

        y_send_base = (1 - my_y) * m_half + my_x * quarter
        my_half_base = my_y * m_half

        dma_send = pltpu.make_async_copy(
            p_ref.at[0, pl.ds(y_send_base, quarter), :], sstage,
            local_sems.at[0],
        )
        dma_local = pltpu.make_async_copy(
            p_ref.at[0, pl.ds(my_half_base, m_half), :], lbuf,
            local_sems.at[1],
        )
        dma_send.start()
        dma_local.start()

        dma_send.wait()
        for c in range(N_CHUNKS):
            sl = pl.ds(c * ch, ch)
            ysend[sl] = sstage[sl].astype(jnp.bfloat16)

        pl.semaphore_wait(barrier_sem, 1)

        y_rdmas = []
        for c in range(N_CHUNKS):
            sl = pl.ds(c * ch, ch)
            rdma = pltpu.make_async_remote_copy(
                src_ref=ysend.at[sl],
                dst_ref=yrecv.at[sl],
                send_sem=ysend_sems.at[c],
                recv_sem=yrecv_sems.at[c],
                device_id=y_nbr,
                device_id_type=pl.DeviceIdType.MESH,
            )
            rdma.start()
            y_rdmas.append(rdma)

        dma_local.wait()
        g = g_ref[...][None, :]
        my_q = my_x * quarter
        other_q = (1 - my_x) * quarter

        pl.semaphore_wait(xbar_sem, 1)

        x_rdmas = []
        out_dmas = []
        for c in range(N_CHUNKS):
            sl = pl.ds(c * ch, ch)
            y_rdmas[c].wait_recv()
            rdma = pltpu.make_async_remote_copy(
                src_ref=yrecv.at[sl],
                dst_ref=xrecv.at[sl],
                send_sem=xsend_sems.at[c],
                recv_sem=xrecv_sems.at[c],
                device_id=x_nbr,
                device_id_type=pl.DeviceIdType.MESH,
            )
            rdma.start()
            x_rdmas.append(rdma)
            osl = pl.ds(my_q + c * ch, ch)
            s = lbuf[osl, :] + yrecv[sl].astype(jnp.float32)
            ms = jnp.mean(s * s, axis=-1, keepdims=True)
            obuf[osl, :] = s * lax.rsqrt(ms + 1e-6) * g
            odma = pltpu.make_async_copy(
                obuf.at[osl, :], out_ref.at[osl, :], out_sems.at[c]
            )
            odma.start()
            out_dmas.append(odma)

        for c in range(N_CHUNKS):
            sl = pl.ds(c * ch, ch)
            x_rdmas[c].wait_recv()
            osl = pl.ds(other_q + c * ch, ch)
            s = lbuf[osl, :] + xrecv[sl].astype(jnp.float32)
            ms = jnp.mean(s * s, axis=-1, keepdims=True)
            obuf[osl, :] = s * lax.rsqrt(ms + 1e-6) * g
            odma = pltpu.make_async_copy(
                obuf.at[osl, :], out_ref.at[osl, :], out_sems.at[N_CHUNKS + c]
            )
            odma.start()
            out_dmas.append(odma)

        for c in range(N_CHUNKS):
            y_rdmas[c].wait_send()
            x_rdmas[c].wait_send()
        for odma in out_dmas:
            odma.wait()

    return pl.pallas_call(
        body,
        out_shape=jax.ShapeDtypeStruct((m_half, d), jnp.float32),
        in_specs=[
            pl.BlockSpec(memory_space=pl.ANY),
            pl.BlockSpec(memory_space=pltpu.VMEM),
        ],
        out_specs=pl.BlockSpec(memory_space=pl.ANY),
        scratch_shapes=[
            pltpu.VMEM((quarter, d), jnp.float32),
            pltpu.VMEM((m_half, d), jnp.float32),
            pltpu.VMEM((m_half, d), jnp.float32),
            pltpu.VMEM((quarter, d), jnp.bfloat16),
            pltpu.VMEM((quarter, d), jnp.bfloat16),
            pltpu.VMEM((quarter, d), jnp.bfloat16),
            pltpu.SemaphoreType.REGULAR,
            pltpu.SemaphoreType.DMA((2,)),
            pltpu.SemaphoreType.DMA((N_CHUNKS,)),
            pltpu.SemaphoreType.DMA((N_CHUNKS,)),
            pltpu.SemaphoreType.DMA((N_CHUNKS,)),
            pltpu.SemaphoreType.DMA((N_CHUNKS,)),
            pltpu.SemaphoreType.DMA((2 * N_CHUNKS,)),
        ],
        compiler_params=pltpu.CompilerParams(collective_id=0),
    )(partial, gamma)


# device time: 13078 ns/iter; 1.0255x vs baseline; 1.0255x over previous
import jax
import jax.numpy as jnp
from jax import lax
from jax.experimental import pallas as pl
from jax.experimental.pallas import tpu as pltpu


def kernel(partial, gamma):
    m_half = partial.shape[1] // 2
    d = partial.shape[2]

    def body(p_ref, g_ref, out_ref, send_buf, recv_buf, send_sem, recv_sem):
        my_x = lax.axis_index("x")
        my_y = lax.axis_index("y")
        my_z = lax.axis_index("z")
        nbr = (my_x, 1 - my_y, my_z)

        barrier_sem = pltpu.get_barrier_semaphore()
        pl.semaphore_signal(
            barrier_sem, inc=1, device_id=nbr,
            device_id_type=pl.DeviceIdType.MESH,
        )
        pl.semaphore_wait(barrier_sem, 1)

        other_start = (1 - my_y) * m_half
        send_buf[...] = p_ref[0, pl.ds(other_start, m_half), :].astype(
            jnp.bfloat16
        )
        rdma = pltpu.make_async_remote_copy(
            src_ref=send_buf,
            dst_ref=recv_buf,
            send_sem=send_sem,
            recv_sem=recv_sem,
            device_id=nbr,
            device_id_type=pl.DeviceIdType.MESH,
        )
        rdma.start()
        rdma.wait()

        my_start = my_y * m_half
        local = p_ref[0, pl.ds(my_start, m_half), :]
        y = local + recv_buf[...].astype(jnp.float32)
        ms = jnp.mean(y * y, axis=-1, keepdims=True)
        out_ref[...] = y * lax.rsqrt(ms + 1e-6) * g_ref[...][None, :]

    return pl.pallas_call(
        body,
        out_shape=jax.ShapeDtypeStruct((m_half, d), jnp.float32),
        in_specs=[
            pl.BlockSpec(memory_space=pltpu.VMEM),
            pl.BlockSpec(memory_space=pltpu.VMEM),
        ],
        out_specs=pl.BlockSpec(memory_space=pltpu.VMEM),
        scratch_shapes=[
            pltpu.VMEM((m_half, d), jnp.bfloat16),
            pltpu.VMEM((m_half, d), jnp.bfloat16),
            pltpu.SemaphoreType.DMA,
            pltpu.SemaphoreType.DMA,
        ],
        compiler_params=pltpu.CompilerParams(collective_id=0),
    )(partial, gamma)
